# Initial kernel scaffold; baseline (speedup 1.0000x reference)
import jax
import jax.numpy as jnp
from jax.experimental import pallas as pl


def kernel(state, edge_index, W1, b1, Wl1, bl1, Wl2, bl2, Wl3, bl3):
    raise NotImplementedError("write your pallas kernel here")


# speedup vs baseline: 3.2607x; 3.2607x over previous
import jax
import jax.numpy as jnp
from jax.experimental import pallas as pl
from jax.experimental.pallas import tpu as pltpu

N = 10000
D = 128
ACT = 16
ROW_BLK = 2000


def _xw_body(state_ref, w_ref, out_ref):
    out_ref[...] = jax.lax.dot_general(
        state_ref[...], w_ref[...], (((1,), (0,)), ((), ())),
        preferred_element_type=jnp.float32,
        precision=jax.lax.Precision.HIGHEST,
    )


def _head_body(x_ref, wl1_ref, bl1_ref, wl2_ref, bl2_ref, wl3_ref, bl3_ref, out_ref):
    x = x_ref[...].reshape(-1, ACT, D).sum(axis=1)
    h = jax.nn.relu(
        jax.lax.dot_general(x, wl1_ref[...], (((1,), (0,)), ((), ())),
                            preferred_element_type=jnp.float32,
                            precision=jax.lax.Precision.HIGHEST) + bl1_ref[...])
    h = jax.nn.relu(
        jax.lax.dot_general(h, wl2_ref[...], (((1,), (0,)), ((), ())),
                            preferred_element_type=jnp.float32,
                            precision=jax.lax.Precision.HIGHEST) + bl2_ref[...])
    o = jax.lax.dot_general(h, wl3_ref[...], (((1,), (0,)), ((), ())),
                            preferred_element_type=jnp.float32,
                            precision=jax.lax.Precision.HIGHEST) + bl3_ref[...]
    out_ref[...] = o[:, 0]


def kernel(state, edge_index, W1, b1, Wl1, bl1, Wl2, bl2, Wl3, bl3):
    src, dst = edge_index[0], edge_index[1]

    xw = pl.pallas_call(
        _xw_body,
        grid=(N // ROW_BLK,),
        in_specs=[
            pl.BlockSpec((ROW_BLK, D), lambda i: (i, 0)),
            pl.BlockSpec((D, D), lambda i: (0, 0)),
        ],
        out_specs=pl.BlockSpec((ROW_BLK, D), lambda i: (i, 0)),
        out_shape=jax.ShapeDtypeStruct((N, D), jnp.float32),
    )(state, W1)

    deg = jnp.ones((N,), jnp.float32).at[dst].add(1.0)
    dinv = jax.lax.rsqrt(deg)
    y = dinv[:, None] * xw
    agg = jnp.zeros((N, D), jnp.float32).at[dst].add(y[src])
    x = jax.nn.relu(dinv[:, None] * (agg + y) + b1) + state

    out = pl.pallas_call(
        _head_body,
        out_shape=jax.ShapeDtypeStruct((N // ACT,), jnp.float32),
    )(x, Wl1, bl1, Wl2, bl2, Wl3, bl3)
    return out


# speedup vs baseline: 12.1865x; 3.7374x over previous
import functools

import jax
import jax.numpy as jnp
from jax import lax
from jax.experimental import pallas as pl
from jax.experimental.pallas import tpu as pltpu
from jax.experimental.pallas import tpu_sc as plsc

N = 10000
E = 320000
D = 128
ACT = 16
NC = 2
NS = 16
NW = NC * NS
CHUNK = 128
CPW = 80
E_PAD = NW * CPW * CHUNK
ROWS = 10240
RPW = ROWS // NS
DUMMY = N
ROW_BLK = 2000

_HIGH = jax.lax.Precision.HIGHEST

_mesh = plsc.VectorSubcoreMesh(core_axis_name="c", subcore_axis_name="s")

_Z16 = functools.partial(jnp.zeros, (16,), jnp.float32)


@functools.partial(
    pl.kernel,
    out_type=jax.ShapeDtypeStruct((NC, ROWS), jnp.float32),
    mesh=_mesh,
    scratch_types=[
        pltpu.VMEM((CPW, CHUNK), jnp.int32),
        pltpu.VMEM((CHUNK,), jnp.float32),
        pltpu.VMEM((RPW,), jnp.float32),
        pltpu.VMEM_SHARED((ROWS,), jnp.float32),
        pltpu.SemaphoreType.DMA,
    ],
)
def _deg_kernel(pk_hbm, deg_hbm, idx_v, ones_v, zer_v, acc, sem):
    cid = lax.axis_index("c")
    sid = lax.axis_index("s")
    wid = cid * NS + sid

    for i in range(CHUNK // 16):
        ones_v[pl.ds(i * 16, 16)] = jnp.ones((16,), jnp.float32)

    def _z(i, _):
        zer_v[pl.ds(i * 16, 16)] = _Z16()
        return 0
    lax.fori_loop(0, RPW // 16, _z, 0)

    pltpu.sync_copy(zer_v, acc.at[pl.ds(sid * RPW, RPW)])
    pltpu.sync_copy(pk_hbm.at[pl.ds(wid * CPW, CPW)], idx_v)

    def _unpack(i, _):
        r = i // (CHUNK // 16)
        c = (i % (CHUNK // 16)) * 16
        v = idx_v[r, pl.ds(c, 16)]
        idx_v[r, pl.ds(c, 16)] = lax.bitwise_and(v, 32767)
        return 0
    lax.fori_loop(0, CPW * (CHUNK // 16), _unpack, 0)
    plsc.subcore_barrier()

    _K = 8

    def _scat(g, _):
        for b in range(_K):
            pltpu.async_copy(ones_v, acc.at[idx_v.at[g * _K + b]], sem, add=True)
        for b in range(_K):
            pltpu.make_async_copy(ones_v, acc.at[idx_v.at[g * _K + b]], sem).wait()
        return 0
    lax.fori_loop(0, CPW // _K, _scat, 0)

    plsc.subcore_barrier()
    pltpu.sync_copy(acc.at[pl.ds(sid * RPW, RPW)],
                    deg_hbm.at[cid, pl.ds(sid * RPW, RPW)])


def _xw_body(state_ref, w_ref, xw_ref):
    xw_ref[...] = lax.dot_general(
        state_ref[...], w_ref[...], (((1,), (0,)), ((), ())),
        preferred_element_type=jnp.float32, precision=_HIGH)


def _scale_body(xw_ref, deg_ref, y_ref, dinv_ref):
    deg = deg_ref[0, :N] + deg_ref[1, :N] + 1.0
    dinv = lax.rsqrt(deg)
    y_ref[...] = xw_ref[...] * dinv[:, None]
    dinv_ref[...] = dinv[:, None]


@functools.partial(
    pl.kernel,
    out_type=jax.ShapeDtypeStruct((NC, ROWS, D), jnp.float32),
    mesh=_mesh,
    scratch_types=[
        pltpu.VMEM((CPW, CHUNK), jnp.int32),
        pltpu.VMEM((2, CHUNK), jnp.int32),
        pltpu.VMEM((2, CHUNK), jnp.int32),
        pltpu.VMEM((2, CHUNK, D), jnp.float32),
        pltpu.VMEM_SHARED((ROWS, D), jnp.float32),
        pltpu.SemaphoreType.DMA,
        pltpu.SemaphoreType.DMA,
    ],
)
def _agg_kernel(pk_hbm, y_hbm, out_hbm, pk_v, src_c, dst_c, buf, acc,
                sem0, sem1):
    cid = lax.axis_index("c")
    sid = lax.axis_index("s")
    wid = cid * NS + sid

    @pl.when(cid == 0)
    def _():
        @pl.when(sid < NS - 1)
        def _():
            pltpu.sync_copy(y_hbm.at[pl.ds(sid * 624, 624)],
                            acc.at[pl.ds(sid * 624, 624)])

        @pl.when(sid == NS - 1)
        def _():
            pltpu.sync_copy(y_hbm.at[pl.ds((NS - 1) * 624, N - (NS - 1) * 624)],
                            acc.at[pl.ds((NS - 1) * 624, N - (NS - 1) * 624)])

    @pl.when(cid == 1)
    def _():
        def _z(i, _):
            for c in range(D // 16):
                buf[0, i, pl.ds(c * 16, 16)] = _Z16()
            return 0
        lax.fori_loop(0, CHUNK, _z, 0)
        for k in range(RPW // CHUNK):
            pltpu.sync_copy(buf.at[0],
                            acc.at[pl.ds(sid * RPW + k * CHUNK, CHUNK)])

    pltpu.sync_copy(pk_hbm.at[pl.ds(wid * CPW, CPW)], pk_v)
    plsc.subcore_barrier()

    def _unpack(j, b):
        for c in range(CHUNK // 16):
            v = pk_v[j, pl.ds(c * 16, 16)]
            src_c[b, pl.ds(c * 16, 16)] = lax.shift_right_logical(v, 15)
            dst_c[b, pl.ds(c * 16, 16)] = lax.bitwise_and(v, 32767)

    _unpack(0, 0)
    pltpu.async_copy(y_hbm.at[src_c.at[0]], buf.at[0], sem0)

    def _pair(jp, _):
        j0 = 2 * jp
        _unpack(j0 + 1, 1)
        pltpu.async_copy(y_hbm.at[src_c.at[1]], buf.at[1], sem1)
        pltpu.make_async_copy(y_hbm.at[src_c.at[0]], buf.at[0], sem0).wait()
        pltpu.sync_copy(buf.at[0], acc.at[dst_c.at[0]], add=True)

        @pl.when(jp < CPW // 2 - 1)
        def _():
            _unpack(j0 + 2, 0)
            pltpu.async_copy(y_hbm.at[src_c.at[0]], buf.at[0], sem0)

        pltpu.make_async_copy(y_hbm.at[src_c.at[1]], buf.at[1], sem1).wait()
        pltpu.sync_copy(buf.at[1], acc.at[dst_c.at[1]], add=True)
        return 0
    lax.fori_loop(0, CPW // 2, _pair, 0)

    plsc.subcore_barrier()
    pltpu.sync_copy(acc.at[pl.ds(sid * RPW, RPW)],
                    out_hbm.at[cid, pl.ds(sid * RPW, RPW)])


def _head_body(p_ref, dinv_ref, state_ref, b1_ref,
               wl1_ref, bl1_ref, wl2_ref, bl2_ref, wl3_ref, bl3_ref, out_ref):
    agg = p_ref[0] + p_ref[1]
    x = jax.nn.relu(agg * dinv_ref[...] + b1_ref[...][None, :]) + state_ref[...]
    g = x.reshape(-1, ACT, D).sum(axis=1)
    h = jax.nn.relu(
        lax.dot_general(g, wl1_ref[...], (((1,), (0,)), ((), ())),
                        preferred_element_type=jnp.float32, precision=_HIGH)
        + bl1_ref[...])
    h = jax.nn.relu(
        lax.dot_general(h, wl2_ref[...], (((1,), (0,)), ((), ())),
                        preferred_element_type=jnp.float32, precision=_HIGH)
        + bl2_ref[...])
    o = lax.dot_general(h, wl3_ref[...], (((1,), (0,)), ((), ())),
                        preferred_element_type=jnp.float32, precision=_HIGH)
    out_ref[...] = (o[:, 0] + bl3_ref[...])[None, None, :]


def kernel(state, edge_index, W1, b1, Wl1, bl1, Wl2, bl2, Wl3, bl3):
    src, dst = edge_index[0], edge_index[1]
    pad = E_PAD - E
    code = src * 32768 + dst
    pk = jnp.concatenate([code, jnp.full((pad,), DUMMY, jnp.int32)])
    pk = pk.reshape(E_PAD // CHUNK, CHUNK)

    deg = _deg_kernel(pk)
    xw = pl.pallas_call(
        _xw_body,
        grid=(N // ROW_BLK,),
        in_specs=[
            pl.BlockSpec((ROW_BLK, D), lambda i: (i, 0)),
            pl.BlockSpec((D, D), lambda i: (0, 0)),
        ],
        out_specs=pl.BlockSpec((ROW_BLK, D), lambda i: (i, 0)),
        out_shape=jax.ShapeDtypeStruct((N, D), jnp.float32),
    )(state, W1)
    y, dinv = pl.pallas_call(
        _scale_body,
        out_shape=[jax.ShapeDtypeStruct((N, D), jnp.float32),
                   jax.ShapeDtypeStruct((N, 1), jnp.float32)],
    )(xw, deg)
    p = _agg_kernel(pk, y)
    gb = N // ROW_BLK
    out = pl.pallas_call(
        _head_body,
        grid=(gb,),
        in_specs=[
            pl.BlockSpec((NC, ROW_BLK, D), lambda i: (0, i, 0)),
            pl.BlockSpec((ROW_BLK, 1), lambda i: (i, 0)),
            pl.BlockSpec((ROW_BLK, D), lambda i: (i, 0)),
            pl.BlockSpec((D,), lambda i: (0,)),
            pl.BlockSpec((D, 64), lambda i: (0, 0)),
            pl.BlockSpec((64,), lambda i: (0,)),
            pl.BlockSpec((64, 64), lambda i: (0, 0)),
            pl.BlockSpec((64,), lambda i: (0,)),
            pl.BlockSpec((64, 1), lambda i: (0, 0)),
            pl.BlockSpec((1,), lambda i: (0,)),
        ],
        out_specs=pl.BlockSpec((1, 1, ROW_BLK // ACT), lambda i: (i, 0, 0)),
        out_shape=jax.ShapeDtypeStruct((gb, 1, ROW_BLK // ACT), jnp.float32),
    )(p, dinv, state, b1, Wl1, bl1, Wl2, bl2, Wl3, bl3)
    return out.reshape(N // ACT)


# speedup vs baseline: 12.3906x; 1.0167x over previous
import functools

import jax
import jax.numpy as jnp
from jax import lax
from jax.experimental import pallas as pl
from jax.experimental.pallas import tpu as pltpu
from jax.experimental.pallas import tpu_sc as plsc

N = 10000
E = 320000
D = 128
ACT = 16
NC = 2
NS = 16
NW = NC * NS
CHUNK = 128
CPW = 80
E_PAD = NW * CPW * CHUNK
ROWS = 10240
RPW = ROWS // NS
DUMMY = N
ROW_BLK = 2000

_HIGH = jax.lax.Precision.HIGHEST

_mesh = plsc.VectorSubcoreMesh(core_axis_name="c", subcore_axis_name="s")

_Z16 = functools.partial(jnp.zeros, (16,), jnp.float32)


@functools.partial(
    pl.kernel,
    out_type=jax.ShapeDtypeStruct((NC, ROWS), jnp.float32),
    mesh=_mesh,
    scratch_types=[
        pltpu.VMEM((CPW, CHUNK), jnp.int32),
        pltpu.VMEM((CHUNK,), jnp.float32),
        pltpu.VMEM((RPW,), jnp.float32),
        pltpu.VMEM_SHARED((ROWS,), jnp.float32),
        pltpu.SemaphoreType.DMA,
    ],
)
def _deg_kernel(pk_hbm, deg_hbm, idx_v, ones_v, zer_v, acc, sem):
    cid = lax.axis_index("c")
    sid = lax.axis_index("s")
    wid = cid * NS + sid

    for i in range(CHUNK // 16):
        ones_v[pl.ds(i * 16, 16)] = jnp.ones((16,), jnp.float32)

    def _z(i, _):
        zer_v[pl.ds(i * 16, 16)] = _Z16()
        return 0
    lax.fori_loop(0, RPW // 16, _z, 0)

    pltpu.sync_copy(zer_v, acc.at[pl.ds(sid * RPW, RPW)])
    pltpu.sync_copy(pk_hbm.at[pl.ds(wid * CPW, CPW)], idx_v)

    def _unpack(i, _):
        r = i // (CHUNK // 16)
        c = (i % (CHUNK // 16)) * 16
        v = idx_v[r, pl.ds(c, 16)]
        idx_v[r, pl.ds(c, 16)] = lax.bitwise_and(v, 32767)
        return 0
    lax.fori_loop(0, CPW * (CHUNK // 16), _unpack, 0)
    plsc.subcore_barrier()

    _K = 8

    def _scat(g, _):
        for b in range(_K):
            pltpu.async_copy(ones_v, acc.at[idx_v.at[g * _K + b]], sem, add=True)
        for b in range(_K):
            pltpu.make_async_copy(ones_v, acc.at[idx_v.at[g * _K + b]], sem).wait()
        return 0
    lax.fori_loop(0, CPW // _K, _scat, 0)

    plsc.subcore_barrier()
    pltpu.sync_copy(acc.at[pl.ds(sid * RPW, RPW)],
                    deg_hbm.at[cid, pl.ds(sid * RPW, RPW)])


def _xw_body(state_ref, w_ref, xw_ref):
    xw_ref[...] = lax.dot_general(
        state_ref[...], w_ref[...], (((1,), (0,)), ((), ())),
        preferred_element_type=jnp.float32, precision=_HIGH)


def _scale_body(xw_ref, deg_ref, y_ref, dinv_ref):
    deg = deg_ref[0, :N] + deg_ref[1, :N] + 1.0
    dinv = lax.rsqrt(deg)
    y_ref[...] = xw_ref[...] * dinv[:, None]
    dinv_ref[...] = dinv[:, None]


@functools.partial(
    pl.kernel,
    out_type=jax.ShapeDtypeStruct((NC, ROWS, D), jnp.float32),
    mesh=_mesh,
    scratch_types=[
        pltpu.VMEM((CPW, CHUNK), jnp.int32),
        pltpu.VMEM((2, CHUNK), jnp.int32),
        pltpu.VMEM((2, CHUNK), jnp.int32),
        pltpu.VMEM((2, CHUNK, D), jnp.float32),
        pltpu.VMEM_SHARED((ROWS, D), jnp.float32),
        pltpu.SemaphoreType.DMA,
        pltpu.SemaphoreType.DMA,
    ],
)
def _agg_kernel(pk_hbm, y_hbm, out_hbm, pk_v, src_c, dst_c, buf, acc,
                sem0, sem1):
    cid = lax.axis_index("c")
    sid = lax.axis_index("s")
    wid = cid * NS + sid

    @pl.when(cid == 0)
    def _():
        @pl.when(sid < NS - 1)
        def _():
            pltpu.sync_copy(y_hbm.at[pl.ds(sid * 624, 624)],
                            acc.at[pl.ds(sid * 624, 624)])

        @pl.when(sid == NS - 1)
        def _():
            pltpu.sync_copy(y_hbm.at[pl.ds((NS - 1) * 624, N - (NS - 1) * 624)],
                            acc.at[pl.ds((NS - 1) * 624, N - (NS - 1) * 624)])

    @pl.when(cid == 1)
    def _():
        def _z(i, _):
            for c in range(D // 16):
                buf[0, i, pl.ds(c * 16, 16)] = _Z16()
            return 0
        lax.fori_loop(0, CHUNK, _z, 0)
        for k in range(RPW // CHUNK):
            pltpu.sync_copy(buf.at[0],
                            acc.at[pl.ds(sid * RPW + k * CHUNK, CHUNK)])

    pltpu.sync_copy(pk_hbm.at[pl.ds(wid * CPW, CPW)], pk_v)
    plsc.subcore_barrier()

    def _unpack(j, b):
        for c in range(CHUNK // 16):
            v = pk_v[j, pl.ds(c * 16, 16)]
            src_c[b, pl.ds(c * 16, 16)] = lax.shift_right_logical(v, 15)
            dst_c[b, pl.ds(c * 16, 16)] = lax.bitwise_and(v, 32767)

    _unpack(0, 0)
    pltpu.async_copy(y_hbm.at[src_c.at[0]], buf.at[0], sem0)

    def _pair(jp, _):
        j0 = 2 * jp
        _unpack(j0 + 1, 1)
        pltpu.async_copy(y_hbm.at[src_c.at[1]], buf.at[1], sem1)
        pltpu.make_async_copy(y_hbm.at[src_c.at[0]], buf.at[0], sem0).wait()
        pltpu.sync_copy(buf.at[0], acc.at[dst_c.at[0]], add=True)

        @pl.when(jp < CPW // 2 - 1)
        def _():
            _unpack(j0 + 2, 0)
            pltpu.async_copy(y_hbm.at[src_c.at[0]], buf.at[0], sem0)

        pltpu.make_async_copy(y_hbm.at[src_c.at[1]], buf.at[1], sem1).wait()
        pltpu.sync_copy(buf.at[1], acc.at[dst_c.at[1]], add=True)
        return 0
    lax.fori_loop(0, CPW // 2, _pair, 0)

    plsc.subcore_barrier()
    pltpu.sync_copy(acc.at[pl.ds(sid * RPW, RPW)],
                    out_hbm.at[cid, pl.ds(sid * RPW, RPW)])


def _head_body(p_ref, dinv_ref, state_ref, b1_ref,
               wl1_ref, bl1_ref, wl2_ref, bl2_ref, wl3_ref, bl3_ref, out_ref):
    agg = p_ref[0, :N, :] + p_ref[1, :N, :]
    x = jax.nn.relu(agg * dinv_ref[...] + b1_ref[...][None, :]) + state_ref[...]
    g = x.reshape(N // ACT, ACT, D).sum(axis=1)
    h = jax.nn.relu(
        lax.dot_general(g, wl1_ref[...], (((1,), (0,)), ((), ())),
                        preferred_element_type=jnp.float32, precision=_HIGH)
        + bl1_ref[...])
    h = jax.nn.relu(
        lax.dot_general(h, wl2_ref[...], (((1,), (0,)), ((), ())),
                        preferred_element_type=jnp.float32, precision=_HIGH)
        + bl2_ref[...])
    o = lax.dot_general(h, wl3_ref[...], (((1,), (0,)), ((), ())),
                        preferred_element_type=jnp.float32, precision=_HIGH)
    out_ref[...] = o[:, 0] + bl3_ref[...]


def kernel(state, edge_index, W1, b1, Wl1, bl1, Wl2, bl2, Wl3, bl3):
    src, dst = edge_index[0], edge_index[1]
    pad = E_PAD - E
    code = src * 32768 + dst
    pk = jnp.concatenate([code, jnp.full((pad,), DUMMY, jnp.int32)])
    pk = pk.reshape(E_PAD // CHUNK, CHUNK)

    deg = _deg_kernel(pk)
    xw = pl.pallas_call(
        _xw_body,
        grid=(N // ROW_BLK,),
        in_specs=[
            pl.BlockSpec((ROW_BLK, D), lambda i: (i, 0)),
            pl.BlockSpec((D, D), lambda i: (0, 0)),
        ],
        out_specs=pl.BlockSpec((ROW_BLK, D), lambda i: (i, 0)),
        out_shape=jax.ShapeDtypeStruct((N, D), jnp.float32),
    )(state, W1)
    y, dinv = pl.pallas_call(
        _scale_body,
        out_shape=[jax.ShapeDtypeStruct((N, D), jnp.float32),
                   jax.ShapeDtypeStruct((N, 1), jnp.float32)],
    )(xw, deg)
    p = _agg_kernel(pk, y)
    out = pl.pallas_call(
        _head_body,
        out_shape=jax.ShapeDtypeStruct((N // ACT,), jnp.float32),
    )(p, dinv, state, b1, Wl1, bl1, Wl2, bl2, Wl3, bl3)
    return out
